# baseline (speedup 1.0000x reference)
import functools

import jax
import jax.numpy as jnp
from jax import lax
from jax.experimental import pallas as pl
from jax.experimental.pallas import tpu as pltpu
from jax.experimental.pallas import tpu_sc as plsc

N_NODES = 10000
N_EDGES = 320000
D = 128
L = 16
NC = 2
NS = 16
NW = NC * NS
CHUNK = 128
TRIPS = 80
UNROLL = 8
BODIES = TRIPS // UNROLL
NCHUNK = TRIPS * NW
E_PAD = NCHUNK * CHUNK
ROWS_LO = 624
ROWS_HI = N_NODES - 15 * ROWS_LO
ZROWS = 16


def _sc_partials(x, srcp, dstp, wp):
    mesh = plsc.VectorSubcoreMesh(
        core_axis_name="c", subcore_axis_name="s", num_cores=NC, num_subcores=NS
    )

    @functools.partial(
        pl.kernel,
        out_type=jax.ShapeDtypeStruct((NC, N_NODES, D), jnp.float32),
        mesh=mesh,
        scratch_types=[
            pltpu.VMEM_SHARED((N_NODES, D), jnp.float32),
            pltpu.VMEM((2, UNROLL, CHUNK), jnp.int32),
            pltpu.VMEM((2, UNROLL, CHUNK), jnp.int32),
            pltpu.VMEM((2, UNROLL, CHUNK), jnp.float32),
            pltpu.VMEM((2, CHUNK, D), jnp.float32),
            pltpu.VMEM((ZROWS, D), jnp.float32),
            pltpu.SemaphoreType.DMA,
            pltpu.SemaphoreType.DMA,
            pltpu.SemaphoreType.DMA,
            pltpu.SemaphoreType.DMA,
            pltpu.SemaphoreType.DMA,
        ],
    )
    def k(x_hbm, src_hbm, dst_hbm, w_hbm, parts_hbm, acc, sbuf, dbuf, wbuf,
          rows, zbuf, sg0, sg1, ss0, ss1, sem_i):
        cid = lax.axis_index("c")
        sid = lax.axis_index("s")
        wid = sid * NC + cid
        base_row = sid * ROWS_LO
        sem_g = (sg0, sg1)
        sem_s = (ss0, ss1)

        zeros = jnp.zeros((L,), jnp.float32)
        for r in range(ZROWS):
            for j in range(D // L):
                zbuf[r, pl.ds(j * L, L)] = zeros

        def zcopy(kk, _):
            pltpu.sync_copy(zbuf, acc.at[pl.ds(base_row + kk * ZROWS, ZROWS)])
            return 0

        n_owned = jnp.where(sid == NS - 1, ROWS_HI, ROWS_LO)
        lax.fori_loop(0, n_owned // ZROWS, zcopy, 0)
        plsc.subcore_barrier()

        def issue_idx(body, slot):
            base = wid * TRIPS + body * UNROLL
            pltpu.async_copy(src_hbm.at[pl.ds(base, UNROLL)], sbuf.at[slot],
                             sem_i)
            pltpu.async_copy(dst_hbm.at[pl.ds(base, UNROLL)], dbuf.at[slot],
                             sem_i)
            pltpu.async_copy(w_hbm.at[pl.ds(base, UNROLL)], wbuf.at[slot],
                             sem_i)

        def wait_idx(slot):
            pltpu.make_async_copy(src_hbm.at[pl.ds(0, UNROLL)], sbuf.at[slot],
                                  sem_i).wait()
            pltpu.make_async_copy(dst_hbm.at[pl.ds(0, UNROLL)], dbuf.at[slot],
                                  sem_i).wait()
            pltpu.make_async_copy(w_hbm.at[pl.ds(0, UNROLL)], wbuf.at[slot],
                                  sem_i).wait()

        def scale(pb, kk, p):
            def body(g, _):
                wg = wbuf[pb, kk, pl.ds(g * L, L)]
                for ee in range(L):
                    e = g * L + ee
                    ws = wg[ee]
                    for cc in range(D // L):
                        sl = pl.ds(cc * L, L)
                        rows[p, e, sl] = rows[p, e, sl] * ws
                return 0

            lax.fori_loop(0, CHUNK // L, body, 0)

        issue_idx(0, 0)

        def body_fn(b, _):
            pb = lax.rem(b, 2)

            wait_idx(pb)
            gd = [None] * UNROLL
            sc = [None] * UNROLL
            gd[0] = pltpu.async_copy(x_hbm.at[sbuf.at[pb, 0]], rows.at[0],
                                     sem_g[0])
            for kk in range(UNROLL):
                p = kk % 2
                gd[kk].wait()
                if kk < UNROLL - 1:
                    if kk >= 1:
                        sc[kk - 1].wait()
                    gd[kk + 1] = pltpu.async_copy(
                        x_hbm.at[sbuf.at[pb, kk + 1]], rows.at[1 - p],
                        sem_g[1 - p])
                if kk == 1:
                    @pl.when(b < BODIES - 1)
                    def _():
                        issue_idx(b + 1, 1 - pb)
                scale(pb, kk, p)
                sc[kk] = pltpu.async_copy(rows.at[p], acc.at[dbuf.at[pb, kk]],
                                          sem_s[p], add=True)
            sc[UNROLL - 2].wait()
            sc[UNROLL - 1].wait()
            return 0

        lax.fori_loop(0, BODIES, body_fn, 0)
        plsc.subcore_barrier()

        @pl.when(sid < NS - 1)
        def _():
            pltpu.sync_copy(
                acc.at[pl.ds(base_row, ROWS_LO)],
                parts_hbm.at[cid, pl.ds(base_row, ROWS_LO)],
            )

        @pl.when(sid == NS - 1)
        def _():
            pltpu.sync_copy(
                acc.at[pl.ds(15 * ROWS_LO, ROWS_HI)],
                parts_hbm.at[cid, pl.ds(15 * ROWS_LO, ROWS_HI)],
            )

    return k(x, srcp, dstp, wp)


def _combine(p0, p1):
    def body(a_ref, b_ref, o_ref):
        o_ref[...] = jnp.clip(a_ref[...] + b_ref[...], -20.0, 20.0)

    blk = 1000
    spec = pl.BlockSpec((blk, D), lambda i: (i, 0))
    return pl.pallas_call(
        body,
        grid=(N_NODES // blk,),
        in_specs=[spec, spec],
        out_specs=spec,
        out_shape=jax.ShapeDtypeStruct((N_NODES, D), jnp.float32),
    )(p0, p1)


def kernel(t, x, edge_index, edge_weight):
    pad = E_PAD - N_EDGES
    srcp = jnp.concatenate(
        [edge_index[1], jnp.zeros((pad,), jnp.int32)]).reshape(NCHUNK, CHUNK)
    dstp = jnp.concatenate(
        [edge_index[0], jnp.zeros((pad,), jnp.int32)]).reshape(NCHUNK, CHUNK)
    wp = jnp.concatenate(
        [edge_weight, jnp.zeros((pad,), jnp.float32)]).reshape(NCHUNK, CHUNK)
    parts = _sc_partials(x, srcp, dstp, wp)
    return _combine(parts[0], parts[1])


# speedup vs baseline: 2.6403x; 2.6403x over previous
import functools

import jax
import jax.numpy as jnp
from jax import lax
from jax.experimental import pallas as pl
from jax.experimental.pallas import tpu as pltpu
from jax.experimental.pallas import tpu_sc as plsc

N_NODES = 10000
N_EDGES = 320000
D = 128
L = 16
NC = 2
NS = 16
NW = NC * NS
SUB = 128
CHUNK = 256
NSUB = CHUNK // SUB
NCHUNK = N_EDGES // CHUNK
TRIPS = (NCHUNK + NW - 1) // NW
PAIRS = TRIPS // 2
ROWS_LO = 624
ROWS_HI = N_NODES - 15 * ROWS_LO
ZROWS = 16


def _sc_partials(x, packed, pw):
    mesh = plsc.VectorSubcoreMesh(
        core_axis_name="c", subcore_axis_name="s", num_cores=NC, num_subcores=NS
    )

    @functools.partial(
        pl.kernel,
        out_type=jax.ShapeDtypeStruct((NC, N_NODES, D), jnp.float32),
        mesh=mesh,
        scratch_types=[
            pltpu.VMEM_SHARED((N_NODES, D), jnp.float32),
            pltpu.VMEM((2, 2 * NSUB, SUB), jnp.int32),
            pltpu.VMEM((2, NSUB, SUB), jnp.float32),
            pltpu.VMEM((CHUNK, D), jnp.float32),
            pltpu.VMEM((ZROWS, D), jnp.float32),
            pltpu.SemaphoreType.DMA,
            pltpu.SemaphoreType.DMA,
            pltpu.SemaphoreType.DMA,
            pltpu.SemaphoreType.DMA,
            pltpu.SemaphoreType.DMA,
        ],
    )
    def k(x_hbm, pk_hbm, pw_hbm, parts_hbm, acc, pbuf, pwbuf, rows, zbuf,
          sem_g0, sem_g1, sem_i0, sem_i1, sem_sc):
        cid = lax.axis_index("c")
        sid = lax.axis_index("s")
        wid = sid * NC + cid
        base_row = sid * ROWS_LO

        zeros = jnp.zeros((L,), jnp.float32)
        for r in range(ZROWS):
            for j in range(D // L):
                zbuf[r, pl.ds(j * L, L)] = zeros

        def zcopy(kk, _):
            pltpu.sync_copy(zbuf, acc.at[pl.ds(base_row + kk * ZROWS, ZROWS)])
            return 0

        n_owned = jnp.where(sid == NS - 1, ROWS_HI, ROWS_LO)
        lax.fori_loop(0, n_owned // ZROWS, zcopy, 0)
        plsc.subcore_barrier()

        my_trips = jnp.where(wid < NCHUNK - (TRIPS - 1) * NW, TRIPS, TRIPS - 1)
        sems = (sem_i0, sem_i1)

        def issue_idx(trip, pb):
            c = trip * NW + wid
            pltpu.async_copy(pk_hbm.at[c], pbuf.at[pb], sems[pb])
            pltpu.async_copy(pw_hbm.at[c], pwbuf.at[pb], sems[pb])

        def wait_idx(pb):
            pltpu.make_async_copy(pk_hbm.at[0], pbuf.at[pb], sems[pb]).wait()
            pltpu.make_async_copy(pw_hbm.at[0], pwbuf.at[pb], sems[pb]).wait()

        def do_trip(trip, pb):
            wait_idx(pb)
            sem_g = (sem_g0, sem_g1)
            gs = [
                pltpu.async_copy(x_hbm.at[pbuf.at[pb, j]],
                                 rows.at[pl.ds(j * SUB, SUB)], sem_g[j])
                for j in range(NSUB)
            ]

            scs = []
            for j in range(NSUB):
                gs[j].wait()

                def scale(g, _):
                    wg = pwbuf[pb, j, pl.ds(g * L, L)]
                    for ee in range(L):
                        e = j * SUB + g * L + ee
                        ws = wg[ee]
                        for q in range(D // L):
                            sl = pl.ds(q * L, L)
                            rows[e, sl] = rows[e, sl] * ws
                    return 0

                lax.fori_loop(0, SUB // L, scale, 0)
                scs.append(
                    pltpu.async_copy(rows.at[pl.ds(j * SUB, SUB)],
                                     acc.at[pbuf.at[pb, NSUB + j]], sem_sc,
                                     add=True)
                )

            for sdesc in scs:
                sdesc.wait()

            @pl.when(trip + 2 < my_trips)
            def _():
                issue_idx(trip + 2, pb)

        issue_idx(0, 0)

        @pl.when(1 < my_trips)
        def _():
            issue_idx(1, 1)

        def pair_body(i2, _):
            t = 2 * i2

            @pl.when(t < my_trips)
            def _():
                do_trip(t, 0)

            @pl.when(t + 1 < my_trips)
            def _():
                do_trip(t + 1, 1)

            return 0

        lax.fori_loop(0, PAIRS, pair_body, 0)
        plsc.subcore_barrier()

        @pl.when(sid < NS - 1)
        def _():
            pltpu.sync_copy(
                acc.at[pl.ds(base_row, ROWS_LO)],
                parts_hbm.at[cid, pl.ds(base_row, ROWS_LO)],
            )

        @pl.when(sid == NS - 1)
        def _():
            pltpu.sync_copy(
                acc.at[pl.ds(15 * ROWS_LO, ROWS_HI)],
                parts_hbm.at[cid, pl.ds(15 * ROWS_LO, ROWS_HI)],
            )

    return k(x, packed, pw)


def _combine(p0, p1):
    def body(a_ref, b_ref, o_ref):
        o_ref[...] = jnp.clip(a_ref[...] + b_ref[...], -20.0, 20.0)

    blk = 1000
    spec = pl.BlockSpec((blk, D), lambda i: (i, 0))
    return pl.pallas_call(
        body,
        grid=(N_NODES // blk,),
        in_specs=[spec, spec],
        out_specs=spec,
        out_shape=jax.ShapeDtypeStruct((N_NODES, D), jnp.float32),
    )(p0, p1)


def kernel(t, x, edge_index, edge_weight):
    src = edge_index[1].reshape(NCHUNK, NSUB, SUB)
    dst = edge_index[0].reshape(NCHUNK, NSUB, SUB)
    pw = edge_weight.reshape(NCHUNK, NSUB, SUB)
    packed = jnp.concatenate([src, dst], axis=1)
    parts = _sc_partials(x, packed, pw)
    return _combine(parts[0], parts[1])
